# Initial kernel scaffold; baseline (speedup 1.0000x reference)
import jax
import jax.numpy as jnp
from jax.experimental import pallas as pl


def kernel(x, edge_index, edge_weight, weight, root, bias, wa, ba, motif_w, motif_b):
    raise NotImplementedError("write your pallas kernel here")


# speedup vs baseline: 2.9296x; 2.9296x over previous
import functools

import jax
import jax.numpy as jnp
from jax import lax
from jax.experimental import pallas as pl
from jax.experimental.pallas import tpu as pltpu
from jax.experimental.pallas import tpu_sc as plsc

N = 10000
D = 128
CD = 64
E = 320000
NMOTIF = 13

NCORES = 2
NSUB = 16
NTILES = NCORES * NSUB
CHUNK = 128
NCHUNKS = -(-E // (NTILES * CHUNK))
EPT = NCHUNKS * CHUNK
EPAD = EPT * NTILES
NP = 10240
RPT = NP // NSUB


def _conv_body(ng, xx, src, dst, ew, out, src_v, dst_v, ew_v, rows_v, zero_v,
               acc, sem):
    cid = lax.axis_index("c")
    sid = lax.axis_index("s")
    ebase = (cid * NSUB + sid) * EPT

    z16 = jnp.zeros((16,), jnp.float32)
    for r in range(16):
        for c in range(D // 16):
            zero_v[r, pl.ds(c * 16, 16)] = z16

    def graph_body(g, carry):
        def zero_body(z, carry0):
            pltpu.sync_copy(zero_v, acc.at[pl.ds(sid * RPT + z * 16, 16)])
            return carry0

        lax.fori_loop(0, RPT // 16, zero_body, 0)
        plsc.subcore_barrier()

        def chunk_body(j, carry2):
            off = ebase + j * CHUNK
            pltpu.sync_copy(src.at[g, 0, pl.ds(off, CHUNK)], src_v)
            pltpu.sync_copy(dst.at[g, 0, pl.ds(off, CHUNK)], dst_v)
            pltpu.sync_copy(ew.at[g, 0, pl.ds(off, CHUNK)], ew_v)
            pltpu.async_copy(xx.at[src_v], rows_v, sem).wait()

            def group_body(gi, carry3):
                wg = ew_v[pl.ds(gi * 16, 16)]
                for t in range(16):
                    e = gi * 16 + t
                    w = lax.gather(
                        wg, jnp.full((16, 1), t, jnp.int32),
                        lax.GatherDimensionNumbers(
                            offset_dims=(), collapsed_slice_dims=(0,),
                            start_index_map=(0,)),
                        slice_sizes=(1,),
                        mode=lax.GatherScatterMode.PROMISE_IN_BOUNDS)
                    for c in range(D // 16):
                        rows_v[e, pl.ds(c * 16, 16)] = (
                            rows_v[e, pl.ds(c * 16, 16)] * w)
                return carry3

            lax.fori_loop(0, CHUNK // 16, group_body, 0)
            pltpu.sync_copy(rows_v, acc.at[dst_v], add=True)
            return carry2

        lax.fori_loop(0, NCHUNKS, chunk_body, 0)
        plsc.subcore_barrier()
        pltpu.sync_copy(acc.at[pl.ds(sid * RPT, RPT)],
                        out.at[cid, g, pl.ds(sid * RPT, RPT)])
        return carry

    lax.fori_loop(0, ng, graph_body, 0)


@functools.lru_cache(maxsize=None)
def _make_conv(ng):
    mesh = plsc.VectorSubcoreMesh(core_axis_name="c", subcore_axis_name="s")
    return pl.kernel(
        functools.partial(_conv_body, ng),
        out_type=jax.ShapeDtypeStruct((NCORES, ng, NP, D), jnp.float32),
        mesh=mesh,
        scratch_types=[
            pltpu.VMEM((CHUNK,), jnp.int32),
            pltpu.VMEM((CHUNK,), jnp.int32),
            pltpu.VMEM((CHUNK,), jnp.float32),
            pltpu.VMEM((CHUNK, D), jnp.float32),
            pltpu.VMEM((16, D), jnp.float32),
            pltpu.VMEM_SHARED((NP, D), jnp.float32),
            pltpu.SemaphoreType.DMA,
        ],
    )


R_H = 1000


def _h_body(p_ref, x_ref, w_ref, r_ref, b_ref, o_ref):
    agg = p_ref[0] + p_ref[1]
    o_ref[...] = (
        jnp.dot(agg, w_ref[...], preferred_element_type=jnp.float32)
        + jnp.dot(x_ref[...], r_ref[...], preferred_element_type=jnp.float32)
        + b_ref[...]
    )


R_A = 200


def _att_body(h_ref, p_ref, wb_ref, bc_ref, wa_ref, ba_ref, o_ref):
    f32 = jnp.float32
    wa = wa_ref[...]
    c = jnp.dot(h_ref[...], wb_ref[0:D], preferred_element_type=f32)
    mws = []
    for j in range(NMOTIF):
        mj = p_ref[0, j] + p_ref[1, j]
        c = c + jnp.dot(mj, wb_ref[D * (j + 1):D * (j + 2)],
                        preferred_element_type=f32)
        mws.append(jnp.dot(mj, wa, preferred_element_type=f32))
    c = c + bc_ref[...]
    mw = jnp.concatenate(mws, axis=1) + ba_ref[...]
    att = jnp.tanh(jnp.sum((mw * c).reshape(R_A, NMOTIF, CD), axis=2))
    diff = (mw - c).reshape(R_A, NMOTIF, CD)
    o_ref[...] = (att[:, :, None] * diff).reshape(R_A, NMOTIF * CD)


def _build_wbig(motif_w):
    cols = []
    zblk = jnp.zeros((D, CD), jnp.float32)
    for i in range(1, NMOTIF + 1):
        wm = motif_w[i - 1]
        cols.append(jnp.concatenate([wm[: i * D], zblk, wm[i * D:]], axis=0))
    return jnp.concatenate(cols, axis=1)


def kernel(x, edge_index, edge_weight, weight, root, bias, wa, ba, motif_w,
           motif_b):
    src = jnp.pad(edge_index[:, 0, :], ((0, 0), (0, EPAD - E)))[:, None, :]
    dst = jnp.pad(edge_index[:, 1, :], ((0, 0), (0, EPAD - E)))[:, None, :]
    ew = jnp.pad(edge_weight, ((0, 0), (0, EPAD - E)))[:, None, :]

    p0 = _make_conv(1)(x, src[:1], dst[:1], ew[:1])[:, 0, :N, :]

    h = pl.pallas_call(
        _h_body,
        grid=(N // R_H,),
        in_specs=[
            pl.BlockSpec((NCORES, R_H, D), lambda i: (0, i, 0)),
            pl.BlockSpec((R_H, D), lambda i: (i, 0)),
            pl.BlockSpec((D, D), lambda i: (0, 0)),
            pl.BlockSpec((D, D), lambda i: (0, 0)),
            pl.BlockSpec((1, D), lambda i: (0, 0)),
        ],
        out_specs=pl.BlockSpec((R_H, D), lambda i: (i, 0)),
        out_shape=jax.ShapeDtypeStruct((N, D), jnp.float32),
    )(p0, x, weight, root, bias[None, :])

    p = _make_conv(NMOTIF)(h, src[1:], dst[1:], ew[1:])[:, :, :N, :]

    wbig = _build_wbig(motif_w)
    bcat = motif_b.reshape(1, NMOTIF * CD)
    batile = jnp.tile(ba, NMOTIF)[None, :]

    out = pl.pallas_call(
        _att_body,
        grid=(N // R_A,),
        in_specs=[
            pl.BlockSpec((R_A, D), lambda i: (i, 0)),
            pl.BlockSpec((NCORES, NMOTIF, R_A, D), lambda i: (0, 0, i, 0)),
            pl.BlockSpec(((NMOTIF + 1) * D, NMOTIF * CD), lambda i: (0, 0)),
            pl.BlockSpec((1, NMOTIF * CD), lambda i: (0, 0)),
            pl.BlockSpec((D, CD), lambda i: (0, 0)),
            pl.BlockSpec((1, NMOTIF * CD), lambda i: (0, 0)),
        ],
        out_specs=pl.BlockSpec((R_A, NMOTIF * CD), lambda i: (i, 0)),
        out_shape=jax.ShapeDtypeStruct((N, NMOTIF * CD), jnp.float32),
    )(h, p, wbig, bcat, wa, batile)
    return out
